# baseline (speedup 1.0000x reference)
import jax
import jax.numpy as jnp
from jax.experimental import pallas as pl
from jax.experimental.pallas import tpu as pltpu

_B, _N, _K = 64, 196, 4096
_ROWS = _B * _N
_BLK = 128


def _body(mask_ref, ps_ref, pt_ref, sum_ref, cnt_ref):
    i = pl.program_id(0)

    @pl.when(i == 0)
    def _():
        sum_ref[0, 0] = 0.0
        cnt_ref[0, 0] = 0.0

    m = mask_ref[:, 0:1] > 0.0
    ps = ps_ref[...]
    pt = pt_ref[...]
    safe = jnp.where(m, ps, 1.0)
    s = jnp.sum(pt * jnp.log(safe))
    cnt = jnp.sum(mask_ref[:, 0])
    sum_ref[0, 0] += s
    cnt_ref[0, 0] += cnt


def kernel(ps, pt, bool_masked_pos):
    ps2 = ps.reshape(_ROWS, _K)
    pt2 = pt.reshape(_ROWS, _K)
    maskf = jnp.broadcast_to(
        bool_masked_pos.reshape(_ROWS, 1).astype(jnp.float32), (_ROWS, 128)
    )
    out = pl.pallas_call(
        _body,
        grid=(_ROWS // _BLK,),
        in_specs=[
            pl.BlockSpec((_BLK, 128), lambda i: (i, 0)),
            pl.BlockSpec((_BLK, _K), lambda i: (i, 0)),
            pl.BlockSpec((_BLK, _K), lambda i: (i, 0)),
        ],
        out_specs=[
            pl.BlockSpec(memory_space=pltpu.SMEM),
            pl.BlockSpec(memory_space=pltpu.SMEM),
        ],
        out_shape=[
            jax.ShapeDtypeStruct((1, 1), jnp.float32),
            jax.ShapeDtypeStruct((1, 1), jnp.float32),
        ],
    )(maskf, ps2, pt2)
    s, c = out
    return -s[0, 0] / c[0, 0]


# speedup vs baseline: 1.7901x; 1.7901x over previous
import jax
import jax.numpy as jnp
from jax.experimental import pallas as pl
from jax.experimental.pallas import tpu as pltpu

_B, _N, _K = 64, 196, 4096
_BB = 2


def _body(mask_ref, ps_ref, pt_ref, sum_ref, cnt_ref):
    i = pl.program_id(0)

    @pl.when(i == 0)
    def _():
        sum_ref[0, 0] = 0.0
        cnt_ref[0, 0] = 0.0

    m = mask_ref[:, :, 0:1] > 0.0
    ps = ps_ref[...]
    pt = pt_ref[...]
    safe = jnp.where(m, ps, 1.0)
    s = jnp.sum(pt * jnp.log(safe))
    cnt = jnp.sum(mask_ref[:, :, 0])
    sum_ref[0, 0] += s
    cnt_ref[0, 0] += cnt


def kernel(ps, pt, bool_masked_pos):
    maskf = jnp.broadcast_to(
        bool_masked_pos.astype(jnp.float32)[:, :, None], (_B, _N, 128)
    )
    out = pl.pallas_call(
        _body,
        grid=(_B // _BB,),
        in_specs=[
            pl.BlockSpec((_BB, _N, 128), lambda i: (i, 0, 0)),
            pl.BlockSpec((_BB, _N, _K), lambda i: (i, 0, 0)),
            pl.BlockSpec((_BB, _N, _K), lambda i: (i, 0, 0)),
        ],
        out_specs=[
            pl.BlockSpec(memory_space=pltpu.SMEM),
            pl.BlockSpec(memory_space=pltpu.SMEM),
        ],
        out_shape=[
            jax.ShapeDtypeStruct((1, 1), jnp.float32),
            jax.ShapeDtypeStruct((1, 1), jnp.float32),
        ],
    )(maskf, ps, pt)
    s, c = out
    return -s[0, 0] / c[0, 0]


# speedup vs baseline: 6.3491x; 3.5468x over previous
import jax
import jax.numpy as jnp
from jax.experimental import pallas as pl
from jax.experimental.pallas import tpu as pltpu

_B, _N, _K = 64, 196, 4096
_NB = 4


def _body(mask_ref, ps_ref, pt_ref, sum_ref, cnt_ref):
    i = pl.program_id(0)

    @pl.when(i == 0)
    def _():
        sum_ref[0, 0] = 0.0
        cnt_ref[0, 0] = 0.0

    m = mask_ref[:, :, 0:1] > 0.0
    ps = ps_ref[...]
    pt = pt_ref[...]
    safe = jnp.where(m, ps, 1.0)
    s = jnp.sum(pt * jnp.log(safe))
    cnt = jnp.sum(mask_ref[:, :, 0])
    sum_ref[0, 0] += s
    cnt_ref[0, 0] += cnt


def kernel(ps, pt, bool_masked_pos):
    pst = jnp.transpose(ps, (1, 0, 2))
    ptt = jnp.transpose(pt, (1, 0, 2))
    maskf = jnp.broadcast_to(
        bool_masked_pos.T.astype(jnp.float32)[:, :, None], (_N, _B, 128)
    )
    out = pl.pallas_call(
        _body,
        grid=(_N // _NB,),
        in_specs=[
            pl.BlockSpec((_NB, _B, 128), lambda i: (i, 0, 0)),
            pl.BlockSpec((_NB, _B, _K), lambda i: (i, 0, 0)),
            pl.BlockSpec((_NB, _B, _K), lambda i: (i, 0, 0)),
        ],
        out_specs=[
            pl.BlockSpec(memory_space=pltpu.SMEM),
            pl.BlockSpec(memory_space=pltpu.SMEM),
        ],
        out_shape=[
            jax.ShapeDtypeStruct((1, 1), jnp.float32),
            jax.ShapeDtypeStruct((1, 1), jnp.float32),
        ],
    )(maskf, pst, ptt)
    s, c = out
    return -s[0, 0] / c[0, 0]
